# baseline (device time: 16016 ns/iter reference)
import jax
import jax.numpy as jnp
from jax import lax
from jax.experimental import pallas as pl
from jax.experimental.pallas import tpu as pltpu

HALF = 256
C = 8
CK = HALF // C
FK = CK // 2


def kernel(x):
    m, n = x.shape

    def body(x_ref, out_ref, comm_ref,
             y_send, y_recv, x_send, x_recv, z_send, z_recv):
        my_x = lax.axis_index("x")
        my_y = lax.axis_index("y")
        my_z = lax.axis_index("z")
        nbr_y = (my_x, 1 - my_y, my_z)
        nbr_x = (1 - my_x, my_y, my_z)
        nbr_z = (my_x, my_y, 1 - my_z)

        barrier_sem = pltpu.get_barrier_semaphore()
        for nbr in (nbr_y, nbr_x, nbr_z):
            pl.semaphore_signal(
                barrier_sem, inc=1,
                device_id=nbr, device_id_type=pl.DeviceIdType.MESH,
            )
        pl.semaphore_wait(barrier_sem, 3)

        h = (my_x + my_z) % 2
        base = h * HALF

        y_rdmas = []
        for k in range(C):
            rows = pl.ds(base + k * CK, CK)
            rd = pltpu.make_async_remote_copy(
                src_ref=x_ref.at[rows],
                dst_ref=comm_ref.at[rows],
                send_sem=y_send.at[k],
                recv_sem=y_recv.at[k],
                device_id=nbr_y,
                device_id_type=pl.DeviceIdType.MESH,
            )
            rd.start()
            y_rdmas.append(rd)

        fwd_rdmas = []
        for k in range(C):
            rows = pl.ds(base + k * CK, CK)
            y_rdmas[k].wait_recv()
            for nbr, sem_s, sem_r, off in (
                (nbr_x, x_send, x_recv, 0),
                (nbr_z, z_send, z_recv, FK),
            ):
                frows = pl.ds(base + k * CK + off, FK)
                rd = pltpu.make_async_remote_copy(
                    src_ref=comm_ref.at[frows],
                    dst_ref=comm_ref.at[frows],
                    send_sem=sem_s.at[k],
                    recv_sem=sem_r.at[k],
                    device_id=nbr,
                    device_id_type=pl.DeviceIdType.MESH,
                )
                rd.start()
                fwd_rdmas.append(rd)
            out_ref[rows, :] = x_ref[rows, :] + comm_ref[rows, :]

        obase = (1 - h) * HALF
        for k in range(C):
            orows = pl.ds(obase + k * CK, CK)
            rx = pltpu.make_async_remote_copy(
                src_ref=comm_ref.at[pl.ds(obase + k * CK, FK)],
                dst_ref=comm_ref.at[pl.ds(obase + k * CK, FK)],
                send_sem=x_send.at[k],
                recv_sem=x_recv.at[k],
                device_id=nbr_x,
                device_id_type=pl.DeviceIdType.MESH,
            )
            rz = pltpu.make_async_remote_copy(
                src_ref=comm_ref.at[pl.ds(obase + k * CK + FK, FK)],
                dst_ref=comm_ref.at[pl.ds(obase + k * CK + FK, FK)],
                send_sem=z_send.at[k],
                recv_sem=z_recv.at[k],
                device_id=nbr_z,
                device_id_type=pl.DeviceIdType.MESH,
            )
            rx.wait_recv()
            rz.wait_recv()
            out_ref[orows, :] = x_ref[orows, :] + comm_ref[orows, :]

        for k in range(C):
            y_rdmas[k].wait_send()
        for rd in fwd_rdmas:
            rd.wait_send()

    return pl.pallas_call(
        body,
        out_shape=jax.ShapeDtypeStruct((m, n), x.dtype),
        in_specs=[pl.BlockSpec(memory_space=pltpu.VMEM)],
        out_specs=pl.BlockSpec(memory_space=pltpu.VMEM),
        scratch_shapes=[
            pltpu.VMEM((m, n), x.dtype),
            pltpu.SemaphoreType.DMA((C,)),
            pltpu.SemaphoreType.DMA((C,)),
            pltpu.SemaphoreType.DMA((C,)),
            pltpu.SemaphoreType.DMA((C,)),
            pltpu.SemaphoreType.DMA((C,)),
            pltpu.SemaphoreType.DMA((C,)),
        ],
        compiler_params=pltpu.CompilerParams(collective_id=0),
    )(x)


# device time: 15572 ns/iter; 1.0285x vs baseline; 1.0285x over previous
import jax
import jax.numpy as jnp
from jax import lax
from jax.experimental import pallas as pl
from jax.experimental.pallas import tpu as pltpu

HALF = 256
SIZES = [48, 48, 40, 40, 32, 24, 16, 8]
OFFS = [sum(SIZES[:i]) for i in range(len(SIZES))]
C = len(SIZES)


def kernel(x):
    m, n = x.shape

    def body(x_ref, out_ref, comm_ref, y_send, y_recv, x_send, x_recv):
        my_x = lax.axis_index("x")
        my_y = lax.axis_index("y")
        my_z = lax.axis_index("z")
        nbr_y = (my_x, 1 - my_y, my_z)
        nbr_x = (1 - my_x, my_y, my_z)

        barrier_sem = pltpu.get_barrier_semaphore()
        for nbr in (nbr_y, nbr_x):
            pl.semaphore_signal(
                barrier_sem, inc=1,
                device_id=nbr, device_id_type=pl.DeviceIdType.MESH,
            )
        pl.semaphore_wait(barrier_sem, 2)

        base = my_x * HALF

        y_rdmas = []
        for k in range(C):
            rows = pl.ds(base + OFFS[k], SIZES[k])
            rd = pltpu.make_async_remote_copy(
                src_ref=x_ref.at[rows],
                dst_ref=comm_ref.at[rows],
                send_sem=y_send.at[k],
                recv_sem=y_recv.at[k],
                device_id=nbr_y,
                device_id_type=pl.DeviceIdType.MESH,
            )
            rd.start()
            y_rdmas.append(rd)

        x_rdmas = []
        for k in range(C):
            rows = pl.ds(base + OFFS[k], SIZES[k])
            y_rdmas[k].wait_recv()
            rd = pltpu.make_async_remote_copy(
                src_ref=comm_ref.at[rows],
                dst_ref=comm_ref.at[rows],
                send_sem=x_send.at[k],
                recv_sem=x_recv.at[k],
                device_id=nbr_x,
                device_id_type=pl.DeviceIdType.MESH,
            )
            rd.start()
            x_rdmas.append(rd)
            y_rdmas[k].wait_send()
            out_ref[rows, :] = x_ref[rows, :] + comm_ref[rows, :]

        obase = (1 - my_x) * HALF
        for k in range(C):
            orows = pl.ds(obase + OFFS[k], SIZES[k])
            x_rdmas[k].wait_recv()
            out_ref[orows, :] = x_ref[orows, :] + comm_ref[orows, :]

        for k in range(C):
            x_rdmas[k].wait_send()

    return pl.pallas_call(
        body,
        out_shape=jax.ShapeDtypeStruct((m, n), x.dtype),
        in_specs=[pl.BlockSpec(memory_space=pltpu.VMEM)],
        out_specs=pl.BlockSpec(memory_space=pltpu.VMEM),
        input_output_aliases={0: 0},
        scratch_shapes=[
            pltpu.VMEM((m, n), x.dtype),
            pltpu.SemaphoreType.DMA((C,)),
            pltpu.SemaphoreType.DMA((C,)),
            pltpu.SemaphoreType.DMA((C,)),
            pltpu.SemaphoreType.DMA((C,)),
        ],
        compiler_params=pltpu.CompilerParams(collective_id=0),
    )(x)


# device time: 15222 ns/iter; 1.0522x vs baseline; 1.0230x over previous
import jax
import jax.numpy as jnp
from jax import lax
from jax.experimental import pallas as pl
from jax.experimental.pallas import tpu as pltpu

HALF = 256
C = 8
CK = HALF // C


def kernel(x):
    m, n = x.shape

    def body(x_ref, out_ref, comm_ref, y_send, y_recv, x_send, x_recv):
        my_x = lax.axis_index("x")
        my_y = lax.axis_index("y")
        my_z = lax.axis_index("z")
        nbr_y = (my_x, 1 - my_y, my_z)
        nbr_x = (1 - my_x, my_y, my_z)

        barrier_sem = pltpu.get_barrier_semaphore()
        for nbr in (nbr_y, nbr_x):
            pl.semaphore_signal(
                barrier_sem, inc=1,
                device_id=nbr, device_id_type=pl.DeviceIdType.MESH,
            )
        pl.semaphore_wait(barrier_sem, 2)

        base = my_x * HALF

        y_rdmas = []
        for k in range(C):
            rows = pl.ds(base + k * CK, CK)
            rd = pltpu.make_async_remote_copy(
                src_ref=x_ref.at[rows],
                dst_ref=comm_ref.at[rows],
                send_sem=y_send.at[k],
                recv_sem=y_recv.at[k],
                device_id=nbr_y,
                device_id_type=pl.DeviceIdType.MESH,
            )
            rd.start()
            y_rdmas.append(rd)

        x_rdmas = []
        for k in range(C):
            rows = pl.ds(base + k * CK, CK)
            y_rdmas[k].wait_recv()
            rd = pltpu.make_async_remote_copy(
                src_ref=comm_ref.at[rows],
                dst_ref=comm_ref.at[rows],
                send_sem=x_send.at[k],
                recv_sem=x_recv.at[k],
                device_id=nbr_x,
                device_id_type=pl.DeviceIdType.MESH,
            )
            rd.start()
            x_rdmas.append(rd)
            out_ref[rows, :] = x_ref[rows, :] + comm_ref[rows, :]

        obase = (1 - my_x) * HALF
        for k in range(C):
            orows = pl.ds(obase + k * CK, CK)
            x_rdmas[k].wait_recv()
            out_ref[orows, :] = x_ref[orows, :] + comm_ref[orows, :]

        for k in range(C):
            y_rdmas[k].wait_send()
            x_rdmas[k].wait_send()

    return pl.pallas_call(
        body,
        out_shape=jax.ShapeDtypeStruct((m, n), x.dtype),
        in_specs=[pl.BlockSpec(memory_space=pltpu.VMEM)],
        out_specs=pl.BlockSpec(memory_space=pltpu.VMEM),
        scratch_shapes=[
            pltpu.VMEM((m, n), x.dtype),
            pltpu.SemaphoreType.DMA((C,)),
            pltpu.SemaphoreType.DMA((C,)),
            pltpu.SemaphoreType.DMA((C,)),
            pltpu.SemaphoreType.DMA((C,)),
        ],
        compiler_params=pltpu.CompilerParams(collective_id=0),
    )(x)
